# baseline (speedup 1.0000x reference)
import functools

import jax
import jax.numpy as jnp
from jax import lax
from jax.experimental import pallas as pl
from jax.experimental.pallas import tpu as pltpu
from jax.experimental.pallas import tpu_sc as plsc

E = 8
K = 2
T = 2048
H = 1024
F = 2048
C = 512
S = E * C
TRASH = S
SROWS = S + 512

NW = 32
TPW = T // NW


def _router_body(x_ref, rk_ref, di0_ref, di1_ref, ci0_ref, ci1_ref,
                 g0_ref, g1_ref):
    x = x_ref[...]
    rk = rk_ref[...]
    logits = jax.lax.dot_general(
        x, rk, (((1,), (0,)), ((), ())),
        precision=lax.Precision.DEFAULT,
        preferred_element_type=jnp.float32)
    m = jnp.max(logits, axis=-1, keepdims=True)
    ex = jnp.exp(logits - m)
    probs = ex / jnp.sum(ex, axis=-1, keepdims=True)

    lane = lax.broadcasted_iota(jnp.int32, (T, E), 1)
    p0v = jnp.max(probs, axis=-1, keepdims=True)
    is0 = probs == p0v
    e0 = jnp.min(jnp.where(is0, lane, E), axis=-1, keepdims=True)
    probs_m = jnp.where(lane == e0, -jnp.inf, probs)
    p1v = jnp.max(probs_m, axis=-1, keepdims=True)
    is1 = probs_m == p1v
    e1 = jnp.min(jnp.where(is1, lane, E), axis=-1, keepdims=True)

    oh0 = (lane == e0).astype(jnp.float32)
    oh1 = (lane == e1).astype(jnp.float32)
    ti = lax.broadcasted_iota(jnp.int32, (T, T), 0)
    tj = lax.broadcasted_iota(jnp.int32, (T, T), 1)
    tril = (tj < ti).astype(jnp.float32)
    cnt0 = jax.lax.dot_general(
        tril, oh0, (((1,), (0,)), ((), ())),
        precision=lax.Precision.HIGHEST,
        preferred_element_type=jnp.float32)
    cnt1 = jax.lax.dot_general(
        tril, oh1, (((1,), (0,)), ((), ())),
        precision=lax.Precision.HIGHEST,
        preferred_element_type=jnp.float32)
    total0 = jnp.sum(oh0, axis=0, keepdims=True)

    p0 = jnp.sum(cnt0 * oh0, axis=-1)
    p1 = jnp.sum((cnt1 + total0) * oh1, axis=-1)
    p0i = p0.astype(jnp.int32)
    p1i = p1.astype(jnp.int32)
    e0f = e0[:, 0]
    e1f = e1[:, 0]
    v0 = p0i < C
    v1 = p1i < C
    flat0 = e0f * C + p0i
    flat1 = e1f * C + p1i
    g0 = jnp.sum(probs * oh0, axis=-1)
    g1 = jnp.sum(probs * oh1, axis=-1)

    tok = lax.broadcasted_iota(jnp.int32, (T,), 0)
    s_safe = jnp.sum(jnp.where(tok == 0, e0f, 0)) * C

    di0_ref[...] = jnp.where(v0, flat0, TRASH)
    di1_ref[...] = jnp.where(v1, flat1, TRASH)
    ci0_ref[...] = jnp.where(v0, flat0, s_safe)
    ci1_ref[...] = jnp.where(v1, flat1, s_safe)
    g0_ref[...] = jnp.where(v0, g0, 0.0)
    g1_ref[...] = jnp.where(v1, g1, 0.0)


def _route(x, rk):
    i32 = jax.ShapeDtypeStruct((T,), jnp.int32)
    f32 = jax.ShapeDtypeStruct((T,), jnp.float32)
    return pl.pallas_call(
        _router_body,
        out_shape=(i32, i32, i32, i32, f32, f32),
    )(x, rk)


def _dispatch_body(x_hbm, di0_hbm, di1_hbm, ei_hbm,
                   rows_v, i0_v, i1_v, sem0, sem1):
    wid = lax.axis_index("s") * 2 + lax.axis_index("c")
    base = wid * TPW
    pltpu.sync_copy(di0_hbm.at[pl.ds(base, TPW)], i0_v)
    pltpu.sync_copy(di1_hbm.at[pl.ds(base, TPW)], i1_v)
    pltpu.sync_copy(x_hbm.at[pl.ds(base, TPW)], rows_v)
    cp0 = pltpu.async_copy(rows_v, ei_hbm.at[i0_v], sem0)
    cp1 = pltpu.async_copy(rows_v, ei_hbm.at[i1_v], sem1)
    cp0.wait()
    cp1.wait()


def _dispatch(x, di0, di1):
    mesh = plsc.VectorSubcoreMesh(core_axis_name="c", subcore_axis_name="s")
    kfn = pl.kernel(
        _dispatch_body,
        out_type=jax.ShapeDtypeStruct((SROWS, H), jnp.float32),
        mesh=mesh,
        scratch_types=[
            pltpu.VMEM((TPW, H), jnp.float32),
            pltpu.VMEM((TPW,), jnp.int32),
            pltpu.VMEM((TPW,), jnp.int32),
            pltpu.SemaphoreType.DMA,
            pltpu.SemaphoreType.DMA,
        ],
        compiler_params=pltpu.CompilerParams(needs_layout_passes=False),
    )
    return kfn(x, di0, di1)


def _ffn_body(xe_ref, w1_ref, w2_ref, out_ref):
    x = xe_ref[0]
    w1 = w1_ref[0]
    w2 = w2_ref[0]
    h = jax.lax.dot_general(
        x, w1, (((1,), (0,)), ((), ())),
        precision=lax.Precision.DEFAULT,
        preferred_element_type=jnp.float32)
    h = jnp.maximum(h, 0.0)
    out_ref[0] = jax.lax.dot_general(
        h, w2, (((1,), (0,)), ((), ())),
        precision=lax.Precision.DEFAULT,
        preferred_element_type=jnp.float32)


def _ffn(ei, w_in, w_out):
    return pl.pallas_call(
        _ffn_body,
        grid=(E,),
        in_specs=[
            pl.BlockSpec((1, C, H), lambda e: (e, 0, 0)),
            pl.BlockSpec((1, H, F), lambda e: (e, 0, 0)),
            pl.BlockSpec((1, F, H), lambda e: (e, 0, 0)),
        ],
        out_specs=pl.BlockSpec((1, C, H), lambda e: (e, 0, 0)),
        out_shape=jax.ShapeDtypeStruct((E, C, H), jnp.float32),
    )(ei.reshape(SROWS // C, C, H)[:E], w_in, w_out)


_HB = TPW // 2


def _combine_body(eo_hbm, ci0_hbm, ci1_hbm, g0_hbm, g1_hbm, out_hbm,
                  r0_v, r1_v, i0_v, i1_v, g0_v, g1_v, sem0, sem1):
    wid = lax.axis_index("s") * 2 + lax.axis_index("c")
    for half in range(2):
        base = wid * TPW + half * _HB
        pltpu.sync_copy(ci0_hbm.at[pl.ds(base, _HB)], i0_v)
        pltpu.sync_copy(ci1_hbm.at[pl.ds(base, _HB)], i1_v)
        pltpu.sync_copy(g0_hbm.at[pl.ds(base, _HB)], g0_v)
        pltpu.sync_copy(g1_hbm.at[pl.ds(base, _HB)], g1_v)
        cp0 = pltpu.async_copy(eo_hbm.at[i0_v], r0_v, sem0)
        cp1 = pltpu.async_copy(eo_hbm.at[i1_v], r1_v, sem1)
        cp0.wait()
        cp1.wait()

        def tok_body(i, _):
            g0 = plsc.load_gather(g0_v, [jnp.full((16,), 0, jnp.int32) + i])
            g1 = plsc.load_gather(g1_v, [jnp.full((16,), 0, jnp.int32) + i])
            for c in range(H // 16):
                row = jnp.full((16,), 0, jnp.int32) + i
                col = c * 16 + lax.iota(jnp.int32, 16)
                a = plsc.load_gather(r0_v, [row, col])
                b = plsc.load_gather(r1_v, [row, col])
                plsc.store_scatter(r0_v, [row, col], a * g0 + b * g1)
            return 0

        lax.fori_loop(0, _HB, tok_body, 0)
        pltpu.sync_copy(r0_v, out_hbm.at[pl.ds(base, _HB)])


def _combine(eo, ci0, ci1, g0, g1):
    mesh = plsc.VectorSubcoreMesh(core_axis_name="c", subcore_axis_name="s")
    kfn = pl.kernel(
        _combine_body,
        out_type=jax.ShapeDtypeStruct((T, H), jnp.float32),
        mesh=mesh,
        scratch_types=[
            pltpu.VMEM((_HB, H), jnp.float32),
            pltpu.VMEM((_HB, H), jnp.float32),
            pltpu.VMEM((_HB,), jnp.int32),
            pltpu.VMEM((_HB,), jnp.int32),
            pltpu.VMEM((_HB,), jnp.float32),
            pltpu.VMEM((_HB,), jnp.float32),
            pltpu.SemaphoreType.DMA,
            pltpu.SemaphoreType.DMA,
        ],
        compiler_params=pltpu.CompilerParams(needs_layout_passes=False),
    )
    return kfn(eo.reshape(S, H), ci0, ci1, g0, g1)


def kernel(token_inputs, router_kernel, w_in, w_out):
    g, t, h = token_inputs.shape
    x = token_inputs.reshape(t, h)
    di0, di1, ci0, ci1, g0, g1 = _route(x, router_kernel)
    ei = _dispatch(x, di0, di1)
    eo = _ffn(ei, w_in, w_out)
    out = _combine(eo, ci0, ci1, g0, g1)
    return out.reshape(g, t, h)


# speedup vs baseline: 1.2487x; 1.2487x over previous
import functools

import jax
import jax.numpy as jnp
from jax import lax
from jax.experimental import pallas as pl
from jax.experimental.pallas import tpu as pltpu
from jax.experimental.pallas import tpu_sc as plsc

E = 8
K = 2
T = 2048
H = 1024
F = 2048
C = 512
S = E * C
TRASH = S
SROWS = S + 512

NW = 32
TPW = T // NW


def _router_body(x_ref, rk_ref, di0_ref, di1_ref, ci0_ref, ci1_ref,
                 g0_ref, g1_ref):
    x = x_ref[...]
    rk = rk_ref[...]
    logits = jax.lax.dot_general(
        x, rk, (((1,), (0,)), ((), ())),
        precision=lax.Precision.DEFAULT,
        preferred_element_type=jnp.float32)
    m = jnp.max(logits, axis=-1, keepdims=True)
    ex = jnp.exp(logits - m)
    probs = ex / jnp.sum(ex, axis=-1, keepdims=True)

    lane = lax.broadcasted_iota(jnp.int32, (T, E), 1)
    p0v = jnp.max(probs, axis=-1, keepdims=True)
    is0 = probs == p0v
    e0 = jnp.min(jnp.where(is0, lane, E), axis=-1, keepdims=True)
    probs_m = jnp.where(lane == e0, -jnp.inf, probs)
    p1v = jnp.max(probs_m, axis=-1, keepdims=True)
    is1 = probs_m == p1v
    e1 = jnp.min(jnp.where(is1, lane, E), axis=-1, keepdims=True)

    oh0 = (lane == e0).astype(jnp.float32)
    oh1 = (lane == e1).astype(jnp.float32)
    ti = lax.broadcasted_iota(jnp.int32, (T, T), 0)
    tj = lax.broadcasted_iota(jnp.int32, (T, T), 1)
    tril = (tj < ti).astype(jnp.float32)
    cnt0 = jax.lax.dot_general(
        tril, oh0, (((1,), (0,)), ((), ())),
        precision=lax.Precision.DEFAULT,
        preferred_element_type=jnp.float32)
    cnt1 = jax.lax.dot_general(
        tril, oh1, (((1,), (0,)), ((), ())),
        precision=lax.Precision.DEFAULT,
        preferred_element_type=jnp.float32)
    total0 = jnp.sum(oh0, axis=0, keepdims=True)

    p0 = jnp.sum(cnt0 * oh0, axis=-1)
    p1 = jnp.sum((cnt1 + total0) * oh1, axis=-1)
    p0i = p0.astype(jnp.int32)
    p1i = p1.astype(jnp.int32)
    e0f = e0[:, 0]
    e1f = e1[:, 0]
    v0 = p0i < C
    v1 = p1i < C
    flat0 = e0f * C + p0i
    flat1 = e1f * C + p1i
    g0 = jnp.sum(probs * oh0, axis=-1)
    g1 = jnp.sum(probs * oh1, axis=-1)

    tok = lax.broadcasted_iota(jnp.int32, (T,), 0)
    s_safe = jnp.sum(jnp.where(tok == 0, e0f, 0)) * C

    di0_ref[...] = jnp.where(v0, flat0, TRASH)
    di1_ref[...] = jnp.where(v1, flat1, TRASH)
    ci0_ref[...] = jnp.where(v0, flat0, s_safe)
    ci1_ref[...] = jnp.where(v1, flat1, s_safe)
    g0_ref[...] = jnp.where(v0, g0, 0.0)
    g1_ref[...] = jnp.where(v1, g1, 0.0)


def _route(x, rk):
    i32 = jax.ShapeDtypeStruct((T,), jnp.int32)
    f32 = jax.ShapeDtypeStruct((T,), jnp.float32)
    return pl.pallas_call(
        _router_body,
        out_shape=(i32, i32, i32, i32, f32, f32),
    )(x, rk)


def _dispatch_body(x_hbm, di0_hbm, di1_hbm, ei_hbm,
                   rows_v, i0_v, i1_v, sem0, sem1):
    wid = lax.axis_index("s") * 2 + lax.axis_index("c")
    base = wid * TPW
    pltpu.sync_copy(di0_hbm.at[pl.ds(base, TPW)], i0_v)
    pltpu.sync_copy(di1_hbm.at[pl.ds(base, TPW)], i1_v)
    pltpu.sync_copy(x_hbm.at[pl.ds(base, TPW)], rows_v)
    cp0 = pltpu.async_copy(rows_v, ei_hbm.at[i0_v], sem0)
    cp1 = pltpu.async_copy(rows_v, ei_hbm.at[i1_v], sem1)
    cp0.wait()
    cp1.wait()


def _dispatch(x, di0, di1):
    mesh = plsc.VectorSubcoreMesh(core_axis_name="c", subcore_axis_name="s")
    kfn = pl.kernel(
        _dispatch_body,
        out_type=jax.ShapeDtypeStruct((SROWS, H), jnp.float32),
        mesh=mesh,
        scratch_types=[
            pltpu.VMEM((TPW, H), jnp.float32),
            pltpu.VMEM((TPW,), jnp.int32),
            pltpu.VMEM((TPW,), jnp.int32),
            pltpu.SemaphoreType.DMA,
            pltpu.SemaphoreType.DMA,
        ],
        compiler_params=pltpu.CompilerParams(needs_layout_passes=False),
    )
    return kfn(x, di0, di1)


_NF = 2
_FT = F // _NF


def _ffn_body(xe_ref, w1_ref, w2_ref, out_ref):
    f = pl.program_id(1)
    x = xe_ref[0]
    w1 = w1_ref[0]
    w2 = w2_ref[0]
    h = jax.lax.dot_general(
        x, w1, (((1,), (0,)), ((), ())),
        precision=lax.Precision.DEFAULT,
        preferred_element_type=jnp.float32)
    h = jnp.maximum(h, 0.0)
    part = jax.lax.dot_general(
        h, w2, (((1,), (0,)), ((), ())),
        precision=lax.Precision.DEFAULT,
        preferred_element_type=jnp.float32)

    @pl.when(f == 0)
    def _():
        out_ref[0] = part

    @pl.when(f != 0)
    def _():
        out_ref[0] += part


def _ffn(ei, w_in, w_out):
    return pl.pallas_call(
        _ffn_body,
        grid=(E, _NF),
        in_specs=[
            pl.BlockSpec((1, C, H), lambda e, f: (e, 0, 0)),
            pl.BlockSpec((1, H, _FT), lambda e, f: (e, 0, f)),
            pl.BlockSpec((1, _FT, H), lambda e, f: (e, f, 0)),
        ],
        out_specs=pl.BlockSpec((1, C, H), lambda e, f: (e, 0, 0)),
        out_shape=jax.ShapeDtypeStruct((E, C, H), jnp.float32),
    )(ei.reshape(SROWS // C, C, H)[:E], w_in, w_out)


_HB = TPW // 2


def _combine_body(eo_hbm, ci0_hbm, ci1_hbm, g0_hbm, g1_hbm, out_hbm,
                  r0_v, r1_v, i0_v, i1_v, g0_v, g1_v, sem0, sem1):
    wid = lax.axis_index("s") * 2 + lax.axis_index("c")
    for half in range(2):
        base = wid * TPW + half * _HB
        pltpu.sync_copy(ci0_hbm.at[pl.ds(base, _HB)], i0_v)
        pltpu.sync_copy(ci1_hbm.at[pl.ds(base, _HB)], i1_v)
        pltpu.sync_copy(g0_hbm.at[pl.ds(base, _HB)], g0_v)
        pltpu.sync_copy(g1_hbm.at[pl.ds(base, _HB)], g1_v)
        cp0 = pltpu.async_copy(eo_hbm.at[i0_v], r0_v, sem0)
        cp1 = pltpu.async_copy(eo_hbm.at[i1_v], r1_v, sem1)
        cp0.wait()
        cp1.wait()

        def tok_body(i, _):
            g0 = plsc.load_gather(g0_v, [jnp.full((16,), 0, jnp.int32) + i])
            g1 = plsc.load_gather(g1_v, [jnp.full((16,), 0, jnp.int32) + i])
            for c in range(H // 16):
                sl = pl.ds(c * 16, 16)
                a = r0_v[i, sl]
                b = r1_v[i, sl]
                r0_v[i, sl] = a * g0 + b * g1
            return 0

        lax.fori_loop(0, _HB, tok_body, 0)
        pltpu.sync_copy(r0_v, out_hbm.at[pl.ds(base, _HB)])


def _combine(eo, ci0, ci1, g0, g1):
    mesh = plsc.VectorSubcoreMesh(core_axis_name="c", subcore_axis_name="s")
    kfn = pl.kernel(
        _combine_body,
        out_type=jax.ShapeDtypeStruct((T, H), jnp.float32),
        mesh=mesh,
        scratch_types=[
            pltpu.VMEM((_HB, H), jnp.float32),
            pltpu.VMEM((_HB, H), jnp.float32),
            pltpu.VMEM((_HB,), jnp.int32),
            pltpu.VMEM((_HB,), jnp.int32),
            pltpu.VMEM((_HB,), jnp.float32),
            pltpu.VMEM((_HB,), jnp.float32),
            pltpu.SemaphoreType.DMA,
            pltpu.SemaphoreType.DMA,
        ],
        compiler_params=pltpu.CompilerParams(needs_layout_passes=False),
    )
    return kfn(eo.reshape(S, H), ci0, ci1, g0, g1)


def kernel(token_inputs, router_kernel, w_in, w_out):
    g, t, h = token_inputs.shape
    x = token_inputs.reshape(t, h)
    di0, di1, ci0, ci1, g0, g1 = _route(x, router_kernel)
    ei = _dispatch(x, di0, di1)
    eo = _ffn(ei, w_in, w_out)
    out = _combine(eo, ci0, ci1, g0, g1)
    return out.reshape(g, t, h)


# speedup vs baseline: 1.3722x; 1.0990x over previous
import functools

import jax
import jax.numpy as jnp
from jax import lax
from jax.experimental import pallas as pl
from jax.experimental.pallas import tpu as pltpu
from jax.experimental.pallas import tpu_sc as plsc

E = 8
K = 2
T = 2048
H = 1024
F = 2048
C = 512
S = E * C
TRASH = S
SROWS = S + 512

NW = 32
TPW = T // NW


def _router_body(x_ref, rk_ref, di0_ref, di1_ref, ci0_ref, ci1_ref,
                 g0_ref, g1_ref):
    x = x_ref[...]
    rk = rk_ref[...]
    logits = jax.lax.dot_general(
        x, rk, (((1,), (0,)), ((), ())),
        precision=lax.Precision.DEFAULT,
        preferred_element_type=jnp.float32)
    m = jnp.max(logits, axis=-1, keepdims=True)
    ex = jnp.exp(logits - m)
    probs = ex / jnp.sum(ex, axis=-1, keepdims=True)

    lane = lax.broadcasted_iota(jnp.int32, (T, E), 1)
    p0v = jnp.max(probs, axis=-1, keepdims=True)
    is0 = probs == p0v
    e0 = jnp.min(jnp.where(is0, lane, E), axis=-1, keepdims=True)
    probs_m = jnp.where(lane == e0, -jnp.inf, probs)
    p1v = jnp.max(probs_m, axis=-1, keepdims=True)
    is1 = probs_m == p1v
    e1 = jnp.min(jnp.where(is1, lane, E), axis=-1, keepdims=True)

    oh0 = (lane == e0).astype(jnp.float32)
    oh1 = (lane == e1).astype(jnp.float32)
    ti = lax.broadcasted_iota(jnp.int32, (T, T), 0)
    tj = lax.broadcasted_iota(jnp.int32, (T, T), 1)
    tril = (tj < ti).astype(jnp.float32)
    cnt0 = jax.lax.dot_general(
        tril, oh0, (((1,), (0,)), ((), ())),
        precision=lax.Precision.DEFAULT,
        preferred_element_type=jnp.float32)
    cnt1 = jax.lax.dot_general(
        tril, oh1, (((1,), (0,)), ((), ())),
        precision=lax.Precision.DEFAULT,
        preferred_element_type=jnp.float32)
    total0 = jnp.sum(oh0, axis=0, keepdims=True)

    p0 = jnp.sum(cnt0 * oh0, axis=-1)
    p1 = jnp.sum((cnt1 + total0) * oh1, axis=-1)
    p0i = p0.astype(jnp.int32)
    p1i = p1.astype(jnp.int32)
    e0f = e0[:, 0]
    e1f = e1[:, 0]
    v0 = p0i < C
    v1 = p1i < C
    flat0 = e0f * C + p0i
    flat1 = e1f * C + p1i
    g0 = jnp.sum(probs * oh0, axis=-1)
    g1 = jnp.sum(probs * oh1, axis=-1)

    tok = lax.broadcasted_iota(jnp.int32, (T,), 0)
    s_safe = jnp.sum(jnp.where(tok == 0, e0f, 0)) * C

    di0_ref[...] = jnp.where(v0, flat0, TRASH)
    di1_ref[...] = jnp.where(v1, flat1, TRASH)
    ci0_ref[...] = jnp.where(v0, flat0, s_safe)
    ci1_ref[...] = jnp.where(v1, flat1, s_safe)
    g0_ref[...] = jnp.where(v0, g0, 0.0)
    g1_ref[...] = jnp.where(v1, g1, 0.0)


def _route(x, rk):
    i32 = jax.ShapeDtypeStruct((T,), jnp.int32)
    f32 = jax.ShapeDtypeStruct((T,), jnp.float32)
    return pl.pallas_call(
        _router_body,
        out_shape=(i32, i32, i32, i32, f32, f32),
    )(x, rk)


def _dispatch_body(x_hbm, di0_hbm, di1_hbm, ei_hbm,
                   rows_v, i0_v, i1_v, sem0, sem1):
    wid = lax.axis_index("s") * 2 + lax.axis_index("c")
    base = wid * TPW
    pltpu.sync_copy(di0_hbm.at[pl.ds(base, TPW)], i0_v)
    pltpu.sync_copy(di1_hbm.at[pl.ds(base, TPW)], i1_v)
    pltpu.sync_copy(x_hbm.at[pl.ds(base, TPW)], rows_v)
    cp0 = pltpu.async_copy(rows_v, ei_hbm.at[i0_v], sem0)
    cp1 = pltpu.async_copy(rows_v, ei_hbm.at[i1_v], sem1)
    cp0.wait()
    cp1.wait()


def _dispatch(x, di0, di1):
    mesh = plsc.VectorSubcoreMesh(core_axis_name="c", subcore_axis_name="s")
    kfn = pl.kernel(
        _dispatch_body,
        out_type=jax.ShapeDtypeStruct((SROWS, H), jnp.float32),
        mesh=mesh,
        scratch_types=[
            pltpu.VMEM((TPW, H), jnp.float32),
            pltpu.VMEM((TPW,), jnp.int32),
            pltpu.VMEM((TPW,), jnp.int32),
            pltpu.SemaphoreType.DMA,
            pltpu.SemaphoreType.DMA,
        ],
        compiler_params=pltpu.CompilerParams(needs_layout_passes=False),
    )
    return kfn(x, di0, di1)


_NF = 2
_FT = F // _NF


def _ffn_body(xe_ref, w1_ref, w2_ref, out_ref):
    f = pl.program_id(1)
    x = xe_ref[...]
    w1 = w1_ref[0]
    w2 = w2_ref[0]
    h = jax.lax.dot_general(
        x, w1, (((1,), (0,)), ((), ())),
        precision=lax.Precision.DEFAULT,
        preferred_element_type=jnp.float32)
    h = jnp.maximum(h, 0.0)
    part = jax.lax.dot_general(
        h, w2, (((1,), (0,)), ((), ())),
        precision=lax.Precision.DEFAULT,
        preferred_element_type=jnp.float32)

    @pl.when(f == 0)
    def _():
        out_ref[...] = part

    @pl.when(f != 0)
    def _():
        out_ref[...] += part


def _ffn(ei, w_in, w_out):
    return pl.pallas_call(
        _ffn_body,
        grid=(E, _NF),
        in_specs=[
            pl.BlockSpec((C, H), lambda e, f: (e, 0)),
            pl.BlockSpec((1, H, _FT), lambda e, f: (e, 0, f)),
            pl.BlockSpec((1, _FT, H), lambda e, f: (e, f, 0)),
        ],
        out_specs=pl.BlockSpec((C, H), lambda e, f: (e, 0)),
        out_shape=jax.ShapeDtypeStruct((S, H), jnp.float32),
    )(ei, w_in, w_out)


_HB = TPW // 2


def _combine_body(eo_hbm, ci0_hbm, ci1_hbm, g0_hbm, g1_hbm, out_hbm,
                  r0_v, r1_v, i0_v, i1_v, g0_v, g1_v, sem0, sem1):
    wid = lax.axis_index("s") * 2 + lax.axis_index("c")
    for half in range(2):
        base = wid * TPW + half * _HB
        pltpu.sync_copy(ci0_hbm.at[pl.ds(base, _HB)], i0_v)
        pltpu.sync_copy(ci1_hbm.at[pl.ds(base, _HB)], i1_v)
        pltpu.sync_copy(g0_hbm.at[pl.ds(base, _HB)], g0_v)
        pltpu.sync_copy(g1_hbm.at[pl.ds(base, _HB)], g1_v)
        cp0 = pltpu.async_copy(eo_hbm.at[i0_v], r0_v, sem0)
        cp1 = pltpu.async_copy(eo_hbm.at[i1_v], r1_v, sem1)
        cp0.wait()
        cp1.wait()

        def tok_body(i, _):
            g0 = plsc.load_gather(g0_v, [jnp.full((16,), 0, jnp.int32) + i])
            g1 = plsc.load_gather(g1_v, [jnp.full((16,), 0, jnp.int32) + i])
            for c in range(H // 16):
                sl = pl.ds(c * 16, 16)
                a = r0_v[i, sl]
                b = r1_v[i, sl]
                r0_v[i, sl] = a * g0 + b * g1
            return 0

        lax.fori_loop(0, _HB, tok_body, 0)
        pltpu.sync_copy(r0_v, out_hbm.at[pl.ds(base, _HB)])


def _combine(eo, ci0, ci1, g0, g1):
    mesh = plsc.VectorSubcoreMesh(core_axis_name="c", subcore_axis_name="s")
    kfn = pl.kernel(
        _combine_body,
        out_type=jax.ShapeDtypeStruct((T, H), jnp.float32),
        mesh=mesh,
        scratch_types=[
            pltpu.VMEM((_HB, H), jnp.float32),
            pltpu.VMEM((_HB, H), jnp.float32),
            pltpu.VMEM((_HB,), jnp.int32),
            pltpu.VMEM((_HB,), jnp.int32),
            pltpu.VMEM((_HB,), jnp.float32),
            pltpu.VMEM((_HB,), jnp.float32),
            pltpu.SemaphoreType.DMA,
            pltpu.SemaphoreType.DMA,
        ],
        compiler_params=pltpu.CompilerParams(needs_layout_passes=False),
    )
    return kfn(eo, ci0, ci1, g0, g1)


def kernel(token_inputs, router_kernel, w_in, w_out):
    g, t, h = token_inputs.shape
    x = token_inputs.reshape(t, h)
    di0, di1, ci0, ci1, g0, g1 = _route(x, router_kernel)
    ei = _dispatch(x, di0, di1)
    eo = _ffn(ei, w_in, w_out)
    out = _combine(eo, ci0, ci1, g0, g1)
    return out.reshape(g, t, h)


# speedup vs baseline: 1.4282x; 1.0408x over previous
import functools

import jax
import jax.numpy as jnp
from jax import lax
from jax.experimental import pallas as pl
from jax.experimental.pallas import tpu as pltpu
from jax.experimental.pallas import tpu_sc as plsc

E = 8
K = 2
T = 2048
H = 1024
F = 2048
C = 512
S = E * C
TRASH = S
SROWS = S + 512

NW = 32
TPW = T // NW


def _router_body(x_ref, rk_ref, di0_ref, di1_ref, ci0_ref, ci1_ref,
                 g0_ref, g1_ref):
    x = x_ref[...]
    rk = rk_ref[...]
    logits = jax.lax.dot_general(
        x, rk, (((1,), (0,)), ((), ())),
        precision=lax.Precision.DEFAULT,
        preferred_element_type=jnp.float32)
    m = jnp.max(logits, axis=-1, keepdims=True)
    ex = jnp.exp(logits - m)
    probs = ex / jnp.sum(ex, axis=-1, keepdims=True)

    lane = lax.broadcasted_iota(jnp.int32, (T, E), 1)
    p0v = jnp.max(probs, axis=-1, keepdims=True)
    is0 = probs == p0v
    e0 = jnp.min(jnp.where(is0, lane, E), axis=-1, keepdims=True)
    probs_m = jnp.where(lane == e0, -jnp.inf, probs)
    p1v = jnp.max(probs_m, axis=-1, keepdims=True)
    is1 = probs_m == p1v
    e1 = jnp.min(jnp.where(is1, lane, E), axis=-1, keepdims=True)

    oh0 = (lane == e0).astype(jnp.float32)
    oh1 = (lane == e1).astype(jnp.float32)
    ti = lax.broadcasted_iota(jnp.int32, (T, T), 0)
    tj = lax.broadcasted_iota(jnp.int32, (T, T), 1)
    tril = (tj < ti).astype(jnp.float32)
    cnt0 = jax.lax.dot_general(
        tril, oh0, (((1,), (0,)), ((), ())),
        precision=lax.Precision.DEFAULT,
        preferred_element_type=jnp.float32)
    cnt1 = jax.lax.dot_general(
        tril, oh1, (((1,), (0,)), ((), ())),
        precision=lax.Precision.DEFAULT,
        preferred_element_type=jnp.float32)
    total0 = jnp.sum(oh0, axis=0, keepdims=True)

    p0 = jnp.sum(cnt0 * oh0, axis=-1)
    p1 = jnp.sum((cnt1 + total0) * oh1, axis=-1)
    p0i = p0.astype(jnp.int32)
    p1i = p1.astype(jnp.int32)
    e0f = e0[:, 0]
    e1f = e1[:, 0]
    v0 = p0i < C
    v1 = p1i < C
    flat0 = e0f * C + p0i
    flat1 = e1f * C + p1i
    g0 = jnp.sum(probs * oh0, axis=-1)
    g1 = jnp.sum(probs * oh1, axis=-1)

    tok = lax.broadcasted_iota(jnp.int32, (T,), 0)
    s_safe = jnp.sum(jnp.where(tok == 0, e0f, 0)) * C

    di0_ref[...] = jnp.where(v0, flat0, TRASH)
    di1_ref[...] = jnp.where(v1, flat1, TRASH)
    ci0_ref[...] = jnp.where(v0, flat0, s_safe)
    ci1_ref[...] = jnp.where(v1, flat1, s_safe)
    g0_ref[...] = jnp.where(v0, g0, 0.0)
    g1_ref[...] = jnp.where(v1, g1, 0.0)


def _route(x, rk):
    i32 = jax.ShapeDtypeStruct((T,), jnp.int32)
    f32 = jax.ShapeDtypeStruct((T,), jnp.float32)
    return pl.pallas_call(
        _router_body,
        out_shape=(i32, i32, i32, i32, f32, f32),
    )(x, rk)


def _dispatch_body(x_hbm, di0_hbm, di1_hbm, ei_hbm,
                   rows_v, i0_v, i1_v, sem0, sem1):
    wid = lax.axis_index("s") * 2 + lax.axis_index("c")
    base = wid * TPW
    pltpu.sync_copy(di0_hbm.at[pl.ds(base, TPW)], i0_v)
    pltpu.sync_copy(di1_hbm.at[pl.ds(base, TPW)], i1_v)
    pltpu.sync_copy(x_hbm.at[pl.ds(base, TPW)], rows_v)
    cp0 = pltpu.async_copy(rows_v, ei_hbm.at[i0_v], sem0)
    cp1 = pltpu.async_copy(rows_v, ei_hbm.at[i1_v], sem1)
    cp0.wait()
    cp1.wait()


def _dispatch(x, di0, di1):
    mesh = plsc.VectorSubcoreMesh(core_axis_name="c", subcore_axis_name="s")
    kfn = pl.kernel(
        _dispatch_body,
        out_type=jax.ShapeDtypeStruct((SROWS, H), jnp.float32),
        mesh=mesh,
        scratch_types=[
            pltpu.VMEM((TPW, H), jnp.float32),
            pltpu.VMEM((TPW,), jnp.int32),
            pltpu.VMEM((TPW,), jnp.int32),
            pltpu.SemaphoreType.DMA,
            pltpu.SemaphoreType.DMA,
        ],
        compiler_params=pltpu.CompilerParams(needs_layout_passes=False),
    )
    return kfn(x, di0, di1)


_NF = 1
_FT = F // _NF


def _ffn_body(xe_ref, w1_ref, w2_ref, out_ref):
    f = pl.program_id(1)
    x = xe_ref[...]
    w1 = w1_ref[0]
    w2 = w2_ref[0]
    h = jax.lax.dot_general(
        x, w1, (((1,), (0,)), ((), ())),
        precision=lax.Precision.DEFAULT,
        preferred_element_type=jnp.float32)
    h = jnp.maximum(h, 0.0)
    part = jax.lax.dot_general(
        h, w2, (((1,), (0,)), ((), ())),
        precision=lax.Precision.DEFAULT,
        preferred_element_type=jnp.float32)

    @pl.when(f == 0)
    def _():
        out_ref[...] = part

    @pl.when(f != 0)
    def _():
        out_ref[...] += part


def _ffn(ei, w_in, w_out):
    return pl.pallas_call(
        _ffn_body,
        grid=(E, _NF),
        in_specs=[
            pl.BlockSpec((C, H), lambda e, f: (e, 0)),
            pl.BlockSpec((1, H, _FT), lambda e, f: (e, 0, f)),
            pl.BlockSpec((1, _FT, H), lambda e, f: (e, f, 0)),
        ],
        out_specs=pl.BlockSpec((C, H), lambda e, f: (e, 0)),
        out_shape=jax.ShapeDtypeStruct((S, H), jnp.float32),
    )(ei, w_in, w_out)


_HB = TPW // 2


def _combine_body(eo_hbm, ci0_hbm, ci1_hbm, g0_hbm, g1_hbm, out_hbm,
                  r0_v, r1_v, i0_v, i1_v, g0_v, g1_v, sem0, sem1):
    wid = lax.axis_index("s") * 2 + lax.axis_index("c")
    for half in range(2):
        base = wid * TPW + half * _HB
        pltpu.sync_copy(ci0_hbm.at[pl.ds(base, _HB)], i0_v)
        pltpu.sync_copy(ci1_hbm.at[pl.ds(base, _HB)], i1_v)
        pltpu.sync_copy(g0_hbm.at[pl.ds(base, _HB)], g0_v)
        pltpu.sync_copy(g1_hbm.at[pl.ds(base, _HB)], g1_v)
        cp0 = pltpu.async_copy(eo_hbm.at[i0_v], r0_v, sem0)
        cp1 = pltpu.async_copy(eo_hbm.at[i1_v], r1_v, sem1)
        cp0.wait()
        cp1.wait()

        def tok_body(i, _):
            g0 = plsc.load_gather(g0_v, [jnp.full((16,), 0, jnp.int32) + i])
            g1 = plsc.load_gather(g1_v, [jnp.full((16,), 0, jnp.int32) + i])
            for c in range(H // 16):
                sl = pl.ds(c * 16, 16)
                a = r0_v[i, sl]
                b = r1_v[i, sl]
                r0_v[i, sl] = a * g0 + b * g1
            return 0

        lax.fori_loop(0, _HB, tok_body, 0)
        pltpu.sync_copy(r0_v, out_hbm.at[pl.ds(base, _HB)])


def _combine(eo, ci0, ci1, g0, g1):
    mesh = plsc.VectorSubcoreMesh(core_axis_name="c", subcore_axis_name="s")
    kfn = pl.kernel(
        _combine_body,
        out_type=jax.ShapeDtypeStruct((T, H), jnp.float32),
        mesh=mesh,
        scratch_types=[
            pltpu.VMEM((_HB, H), jnp.float32),
            pltpu.VMEM((_HB, H), jnp.float32),
            pltpu.VMEM((_HB,), jnp.int32),
            pltpu.VMEM((_HB,), jnp.int32),
            pltpu.VMEM((_HB,), jnp.float32),
            pltpu.VMEM((_HB,), jnp.float32),
            pltpu.SemaphoreType.DMA,
            pltpu.SemaphoreType.DMA,
        ],
        compiler_params=pltpu.CompilerParams(needs_layout_passes=False),
    )
    return kfn(eo, ci0, ci1, g0, g1)


def kernel(token_inputs, router_kernel, w_in, w_out):
    g, t, h = token_inputs.shape
    x = token_inputs.reshape(t, h)
    di0, di1, ci0, ci1, g0, g1 = _route(x, router_kernel)
    ei = _dispatch(x, di0, di1)
    eo = _ffn(ei, w_in, w_out)
    out = _combine(eo, ci0, ci1, g0, g1)
    return out.reshape(g, t, h)
